# baseline (speedup 1.0000x reference)
import functools

import jax
import jax.numpy as jnp
from jax import lax
from jax.experimental import pallas as pl
from jax.experimental.pallas import tpu as pltpu
from jax.experimental.pallas import tpu_sc as plsc

VOCAB = 100000
CTX = 200
DIM = 64
BATCH = 4096
SEQ = 200

N = BATCH * SEQ
NC = 2
NS = 16
NW = NC * NS
R = N // NW
IDXW = 80
IDX_ROWS = R // IDXW
CHUNK = 2 * CTX
IDX_PER_CHUNK = CHUNK // IDXW
NCHUNK = R // CHUNK
NBUF = 3
LANES = 16
DSEG = DIM // LANES

_mesh = plsc.VectorSubcoreMesh(core_axis_name="c", subcore_axis_name="s")


@functools.partial(
    pl.kernel,
    mesh=_mesh,
    compiler_params=pltpu.CompilerParams(use_tc_tiling_on_sc=False),
    out_type=jax.ShapeDtypeStruct((BATCH, SEQ, DIM), jnp.float32),
    scratch_types=[
        pltpu.VMEM((IDX_ROWS, IDXW), jnp.int32),
        pltpu.VMEM((CTX, DIM), jnp.float32),
        pltpu.VMEM((CHUNK, DIM), jnp.float32),
        pltpu.VMEM((CHUNK, DIM), jnp.float32),
        pltpu.VMEM((CHUNK, DIM), jnp.float32),
        pltpu.SemaphoreType.DMA,
        pltpu.SemaphoreType.DMA,
        pltpu.SemaphoreType.DMA,
        pltpu.SemaphoreType.DMA,
        pltpu.SemaphoreType.DMA,
        pltpu.SemaphoreType.DMA,
    ],
)
def _embed(x_hbm, vocab_hbm, pos_hbm, out_hbm,
           idx_v, pos_v, r0, r1, r2, g0, g1, g2, s0, s1, s2):
    cid = lax.axis_index("c")
    sid = lax.axis_index("s")
    wid = sid * NC + cid
    rows = (r0, r1, r2)
    gsem = (g0, g1, g2)
    ssem = (s0, s1, s2)

    def fire_gathers(k, buf, sem):
        for j in range(IDX_PER_CHUNK):
            pltpu.async_copy(
                vocab_hbm.at[idx_v.at[k * IDX_PER_CHUNK + j]],
                buf.at[pl.ds(j * IDXW, IDXW)],
                sem,
            )

    def wait_gathers(buf, sem):
        pltpu.make_async_copy(vocab_hbm.at[pl.ds(0, CHUNK)], buf, sem).wait()

    def fire_scatter(k, buf, sem):
        for j in range(CHUNK // CTX):
            pltpu.async_copy(
                buf.at[pl.ds(j * CTX, CTX)],
                out_hbm.at[wid * (R // CTX) + (CHUNK // CTX) * k + j],
                sem,
            )

    def wait_scatter(buf, sem):
        for j in range(CHUNK // CTX):
            pltpu.make_async_copy(buf.at[pl.ds(0, CTX)], out_hbm.at[0], sem).wait()

    def add_pos(buf):
        def body(i, carry):
            for j in range(DSEG):
                sl = pl.ds(j * LANES, LANES)
                p = pos_v[i, sl]
                buf[i, sl] = buf[i, sl] + p
                buf[i + CTX, sl] = buf[i + CTX, sl] + p
            return carry
        lax.fori_loop(0, CTX, body, jnp.int32(0), unroll=8)

    pltpu.sync_copy(x_hbm.at[pl.ds(wid * IDX_ROWS, IDX_ROWS)], idx_v)
    pltpu.sync_copy(pos_hbm, pos_v)

    fire_gathers(0, rows[0], gsem[0])

    def slot(k, s):
        sn = (s + 1) % NBUF
        wait_gathers(rows[s], gsem[s])

        @pl.when(k >= 2)
        def _():
            wait_scatter(rows[sn], ssem[sn])
        fire_gathers(k + 1, rows[sn], gsem[sn])
        add_pos(rows[s])
        fire_scatter(k, rows[s], ssem[s])

    def round_body(g, carry):
        k = NBUF * g
        slot(k, 0)
        slot(k + 1, 1)
        slot(k + 2, 2)
        return carry

    lax.fori_loop(0, (NCHUNK - 1) // NBUF, round_body, jnp.int32(0),
                  unroll=False)

    k_last = NCHUNK - 1
    sl_last = k_last % NBUF
    wait_gathers(rows[sl_last], gsem[sl_last])
    add_pos(rows[sl_last])
    fire_scatter(k_last, rows[sl_last], ssem[sl_last])
    for s in range(NBUF):
        wait_scatter(rows[s], ssem[s])


def kernel(x_bc, vocab_table, pos_table):
    x_flat = x_bc.astype(jnp.int32).reshape(N // IDXW, IDXW)
    return _embed(x_flat, vocab_table, pos_table)


# speedup vs baseline: 1.0023x; 1.0023x over previous
import functools

import jax
import jax.numpy as jnp
from jax import lax
from jax.experimental import pallas as pl
from jax.experimental.pallas import tpu as pltpu
from jax.experimental.pallas import tpu_sc as plsc

VOCAB = 100000
CTX = 200
DIM = 64
BATCH = 4096
SEQ = 200

N = BATCH * SEQ
NC = 2
NS = 16
NW = NC * NS
R = N // NW
IDXW = 80
IDX_ROWS = R // IDXW
CHUNK = 2 * CTX
IDX_PER_CHUNK = CHUNK // IDXW
NCHUNK = R // CHUNK
NBUF = 3
LANES = 16
DSEG = DIM // LANES

_mesh = plsc.VectorSubcoreMesh(core_axis_name="c", subcore_axis_name="s")


@functools.partial(
    pl.kernel,
    mesh=_mesh,
    compiler_params=pltpu.CompilerParams(use_tc_tiling_on_sc=False),
    out_type=jax.ShapeDtypeStruct((BATCH, SEQ, DIM), jnp.float32),
    scratch_types=[
        pltpu.VMEM((IDX_ROWS, IDXW), jnp.int32),
        pltpu.VMEM((CTX, DIM), jnp.float32),
        pltpu.VMEM((CHUNK, DIM), jnp.float32),
        pltpu.VMEM((CHUNK, DIM), jnp.float32),
        pltpu.VMEM((CHUNK, DIM), jnp.float32),
        pltpu.SemaphoreType.DMA,
        pltpu.SemaphoreType.DMA,
        pltpu.SemaphoreType.DMA,
        pltpu.SemaphoreType.DMA,
        pltpu.SemaphoreType.DMA,
        pltpu.SemaphoreType.DMA,
    ],
)
def _embed(x_hbm, vocab_hbm, pos_hbm, out_hbm,
           idx_v, pos_v, r0, r1, r2, g0, g1, g2, s0, s1, s2):
    cid = lax.axis_index("c")
    sid = lax.axis_index("s")
    wid = sid * NC + cid
    rows = (r0, r1, r2)
    gsem = (g0, g1, g2)
    ssem = (s0, s1, s2)

    def fire_gathers(k, buf, sem):
        for j in range(IDX_PER_CHUNK):
            pltpu.async_copy(
                vocab_hbm.at[idx_v.at[k * IDX_PER_CHUNK + j]],
                buf.at[pl.ds(j * IDXW, IDXW)],
                sem,
            )

    def wait_gathers(buf, sem):
        pltpu.make_async_copy(vocab_hbm.at[pl.ds(0, CHUNK)], buf, sem).wait()

    def fire_scatter(k, buf, sem):
        for j in range(CHUNK // CTX):
            pltpu.async_copy(
                buf.at[pl.ds(j * CTX, CTX)],
                out_hbm.at[wid * (R // CTX) + (CHUNK // CTX) * k + j],
                sem,
            )

    def wait_scatter(buf, sem):
        for j in range(CHUNK // CTX):
            pltpu.make_async_copy(buf.at[pl.ds(0, CTX)], out_hbm.at[0], sem).wait()

    def add_pos(buf):
        def body(i, carry):
            for j in range(DSEG):
                sl = pl.ds(j * LANES, LANES)
                p = pos_v[i, sl]
                buf[i, sl] = buf[i, sl] + p
                buf[i + CTX, sl] = buf[i + CTX, sl] + p
            return carry
        lax.fori_loop(0, CTX, body, jnp.int32(0), unroll=4)

    pltpu.sync_copy(x_hbm.at[pl.ds(wid * IDX_ROWS, IDX_ROWS)], idx_v)
    pltpu.sync_copy(pos_hbm, pos_v)

    fire_gathers(0, rows[0], gsem[0])

    def slot(k, s):
        sn = (s + 1) % NBUF
        wait_gathers(rows[s], gsem[s])

        @pl.when(k >= 2)
        def _():
            wait_scatter(rows[sn], ssem[sn])
        fire_gathers(k + 1, rows[sn], gsem[sn])
        add_pos(rows[s])
        fire_scatter(k, rows[s], ssem[s])

    def round_body(g, carry):
        k = NBUF * g
        slot(k, 0)
        slot(k + 1, 1)
        slot(k + 2, 2)
        return carry

    lax.fori_loop(0, (NCHUNK - 1) // NBUF, round_body, jnp.int32(0),
                  unroll=False)

    k_last = NCHUNK - 1
    sl_last = k_last % NBUF
    wait_gathers(rows[sl_last], gsem[sl_last])
    add_pos(rows[sl_last])
    fire_scatter(k_last, rows[sl_last], ssem[sl_last])
    for s in range(NBUF):
        wait_scatter(rows[s], ssem[s])


def kernel(x_bc, vocab_table, pos_table):
    x_flat = x_bc.astype(jnp.int32).reshape(N // IDXW, IDXW)
    return _embed(x_flat, vocab_table, pos_table)
